# Initial kernel scaffold; baseline (speedup 1.0000x reference)
import jax
import jax.numpy as jnp
from jax.experimental import pallas as pl


def kernel(x, edge_index, W, w_ih, w_hh, b_ih, b_hh, fc_w, fc_b):
    raise NotImplementedError("write your pallas kernel here")


# speedup vs baseline: 2.7643x; 2.7643x over previous
import functools

import jax
import jax.numpy as jnp
from jax import lax
from jax.experimental import pallas as pl
from jax.experimental.pallas import tpu as pltpu
from jax.experimental.pallas import tpu_sc as plsc

NNODES = 10000
DIM = 128
NEDGES = 320000

NCORES = 2
NSUB = 16
NTILES = NCORES * NSUB
BATCH = 128
NBATCH = 80
EPAD = NTILES * NBATCH * BATCH
NPAD = 10240
ROWS_PER_SUB = NPAD // NSUB
ZROWS = 64

ROW_BLK = 1000


def _sc_scatter_body(m_hbm, src_hbm, dst_hbm, out_hbm,
                     src_v, dst_v, rows_v, zbuf, agg_sh, sem):
    c = lax.axis_index("c")
    s = lax.axis_index("s")
    w = c * NSUB + s

    zero16 = jnp.zeros((16,), jnp.float32)

    def zrow(i, _):
        def zcol(j, _):
            zbuf[i, pl.ds(j * 16, 16)] = zero16
            return 0
        return lax.fori_loop(0, DIM // 16, zcol, 0)

    lax.fori_loop(0, ZROWS, zrow, 0)
    for k in range(ROWS_PER_SUB // ZROWS):
        pltpu.sync_copy(zbuf, agg_sh.at[pl.ds(s * ROWS_PER_SUB + k * ZROWS, ZROWS)])

    pltpu.sync_copy(src_hbm.at[w], src_v)
    pltpu.sync_copy(dst_hbm.at[w], dst_v)

    plsc.subcore_barrier()

    def step(j, _):
        pltpu.async_copy(m_hbm.at[src_v.at[j]], rows_v, sem).wait()
        pltpu.sync_copy(rows_v, agg_sh.at[dst_v.at[j]], add=True)
        return 0

    lax.fori_loop(0, NBATCH, step, 0)

    plsc.subcore_barrier()

    pltpu.sync_copy(agg_sh.at[pl.ds(s * ROWS_PER_SUB, ROWS_PER_SUB)],
                    out_hbm.at[c, pl.ds(s * ROWS_PER_SUB, ROWS_PER_SUB)])


@functools.cache
def _make_sc_scatter():
    return pl.kernel(
        _sc_scatter_body,
        mesh=plsc.VectorSubcoreMesh(core_axis_name="c", subcore_axis_name="s",
                                    num_cores=NCORES, num_subcores=NSUB),
        out_type=jax.ShapeDtypeStruct((NCORES, NPAD, DIM), jnp.float32),
        scratch_types=[
            pltpu.VMEM((NBATCH, BATCH), jnp.int32),
            pltpu.VMEM((NBATCH, BATCH), jnp.int32),
            pltpu.VMEM((BATCH, DIM), jnp.float32),
            pltpu.VMEM((ZROWS, DIM), jnp.float32),
            pltpu.VMEM_SHARED((NPAD, DIM), jnp.float32),
            pltpu.SemaphoreType.DMA,
        ],
    )


def _tc_pre_body(h_ref, wi_ref, whh_t_ref, bhh_ref, m_ref, gh_ref):
    h = h_ref[...]
    m_ref[...] = jnp.dot(h, wi_ref[...], preferred_element_type=jnp.float32)
    gh_ref[...] = (jnp.dot(h, whh_t_ref[...], preferred_element_type=jnp.float32)
                   + bhh_ref[...])


def _tc_pre(h, wi, whh_t, bhh):
    nblk = NNODES // ROW_BLK
    return pl.pallas_call(
        _tc_pre_body,
        grid=(nblk,),
        in_specs=[
            pl.BlockSpec((ROW_BLK, DIM), lambda i: (i, 0)),
            pl.BlockSpec((DIM, DIM), lambda i: (0, 0)),
            pl.BlockSpec((DIM, 3 * DIM), lambda i: (0, 0)),
            pl.BlockSpec((1, 3 * DIM), lambda i: (0, 0)),
        ],
        out_specs=[
            pl.BlockSpec((ROW_BLK, DIM), lambda i: (i, 0)),
            pl.BlockSpec((ROW_BLK, 3 * DIM), lambda i: (i, 0)),
        ],
        out_shape=[
            jax.ShapeDtypeStruct((NNODES, DIM), jnp.float32),
            jax.ShapeDtypeStruct((NNODES, 3 * DIM), jnp.float32),
        ],
    )(h, wi, whh_t, bhh)


def _tc_gru_body(agg2_ref, gh_ref, h_ref, wih_t_ref, bih_ref, hout_ref):
    agg = agg2_ref[0] + agg2_ref[1]
    gi = (jnp.dot(agg, wih_t_ref[...], preferred_element_type=jnp.float32)
          + bih_ref[...])
    gh = gh_ref[...]
    h = h_ref[...]
    r = jax.nn.sigmoid(gi[:, :DIM] + gh[:, :DIM])
    z = jax.nn.sigmoid(gi[:, DIM:2 * DIM] + gh[:, DIM:2 * DIM])
    n = jnp.tanh(gi[:, 2 * DIM:] + r * gh[:, 2 * DIM:])
    hout_ref[...] = (1.0 - z) * n + z * h


def _tc_gru(agg2, gh, h, wih_t, bih):
    nblk = NNODES // ROW_BLK
    return pl.pallas_call(
        _tc_gru_body,
        grid=(nblk,),
        in_specs=[
            pl.BlockSpec((NCORES, ROW_BLK, DIM), lambda i: (0, i, 0)),
            pl.BlockSpec((ROW_BLK, 3 * DIM), lambda i: (i, 0)),
            pl.BlockSpec((ROW_BLK, DIM), lambda i: (i, 0)),
            pl.BlockSpec((DIM, 3 * DIM), lambda i: (0, 0)),
            pl.BlockSpec((1, 3 * DIM), lambda i: (0, 0)),
        ],
        out_specs=pl.BlockSpec((ROW_BLK, DIM), lambda i: (i, 0)),
        out_shape=jax.ShapeDtypeStruct((NNODES, DIM), jnp.float32),
    )(agg2, gh, h, wih_t, bih)


def _tc_fin_body(h_ref, fcw_t_ref, fcb_ref, o_ref):
    o_ref[...] = (jnp.dot(jnp.maximum(h_ref[...], 0.0), fcw_t_ref[...],
                          preferred_element_type=jnp.float32)
                  + fcb_ref[...])


def _tc_fin(h, fcw_t, fcb):
    nblk = NNODES // ROW_BLK
    return pl.pallas_call(
        _tc_fin_body,
        grid=(nblk,),
        in_specs=[
            pl.BlockSpec((ROW_BLK, DIM), lambda i: (i, 0)),
            pl.BlockSpec((DIM, DIM), lambda i: (0, 0)),
            pl.BlockSpec((1, DIM), lambda i: (0, 0)),
        ],
        out_specs=pl.BlockSpec((ROW_BLK, DIM), lambda i: (i, 0)),
        out_shape=jax.ShapeDtypeStruct((NNODES, DIM), jnp.float32),
    )(h, fcw_t, fcb)


def kernel(x, edge_index, W, w_ih, w_hh, b_ih, b_hh, fc_w, fc_b):
    src = edge_index[0]
    dst = edge_index[1]
    pad = EPAD - NEDGES
    src_p = jnp.concatenate([src, jnp.zeros((pad,), jnp.int32)])
    dst_p = jnp.concatenate([dst, jnp.full((pad,), NNODES, jnp.int32)])
    src_p = src_p.reshape(NTILES, NBATCH, BATCH)
    dst_p = dst_p.reshape(NTILES, NBATCH, BATCH)

    whh_t = w_hh.T
    wih_t = w_ih.T
    fcw_t = fc_w.T
    bhh = b_hh.reshape(1, -1)
    bih = b_ih.reshape(1, -1)
    fcb = fc_b.reshape(1, -1)

    h = x
    for i in range(W.shape[0]):
        m, gh = _tc_pre(h, W[i], whh_t, bhh)
        agg2 = _make_sc_scatter()(m, src_p, dst_p)
        h = _tc_gru(agg2, gh, h, wih_t, bih)
    return _tc_fin(h, fcw_t, fcb)


# speedup vs baseline: 4.7070x; 1.7028x over previous
import functools

import jax
import jax.numpy as jnp
from jax import lax
from jax.experimental import pallas as pl
from jax.experimental.pallas import tpu as pltpu
from jax.experimental.pallas import tpu_sc as plsc

NNODES = 10000
DIM = 128
NEDGES = 320000

NCORES = 2
NSUB = 16
HDIM = DIM // NCORES
BATCH = 128
NBATCH = 160
EPAD = NSUB * NBATCH * BATCH
NPAD = 10240
ROWS_PER_SUB = NPAD // NSUB
ZROWS = 64

ROW_BLK = 1000


NBUF = 5
SDEPTH = 2


def _sc_scatter_body(m_hbm, edges_hbm, out_hbm,
                     src_v, dst_v, *rest):
    rows = rest[:NBUF]
    zbuf, agg_sh, gsem, ssem = rest[NBUF:]
    c = lax.axis_index("c")
    s = lax.axis_index("s")

    zero16 = jnp.zeros((16,), jnp.float32)

    def zrow(i, _):
        def zcol(j, _):
            zbuf[i, pl.ds(j * 16, 16)] = zero16
            return 0
        return lax.fori_loop(0, DIM // 16, zcol, 0)

    lax.fori_loop(0, ZROWS, zrow, 0)
    for k in range(ROWS_PER_SUB // ZROWS):
        pltpu.sync_copy(zbuf, agg_sh.at[pl.ds(s * ROWS_PER_SUB + k * ZROWS, ZROWS)])

    pltpu.sync_copy(edges_hbm.at[0, s], src_v)
    pltpu.sync_copy(edges_hbm.at[1, s], dst_v)

    plsc.subcore_barrier()

    mc = m_hbm.at[c]

    def fire_g(j, b):
        pltpu.async_copy(mc.at[src_v.at[j]], rows[b], gsem)

    def wait_g(j, b):
        pltpu.make_async_copy(mc.at[src_v.at[j]], rows[b], gsem).wait()

    def fire_s(j, b):
        pltpu.async_copy(rows[b], agg_sh.at[dst_v.at[j]], ssem, add=True)

    def wait_s(j, b):
        pltpu.make_async_copy(rows[b], agg_sh.at[dst_v.at[j]], ssem).wait()

    def slot(j, full):
        b = j % NBUF
        if full:
            jd = j - SDEPTH
            wait_s(jd, jd % NBUF)
            fire_g(jd + NBUF, jd % NBUF)
        wait_g(j, b)
        fire_s(j, b)

    for b in range(NBUF):
        fire_g(b, b)
    for j in range(SDEPTH):
        slot(j, full=False)

    def body(i, _):
        jbase = SDEPTH + i * NBUF
        for u in range(NBUF):
            j = jbase + u
            b = (SDEPTH + u) % NBUF
            bd = u
            wait_s(j - SDEPTH, bd)
            fire_g(j - SDEPTH + NBUF, bd)
            wait_g(j, b)
            fire_s(j, b)
        return 0

    nmid = (NBATCH - NBUF - SDEPTH) // NBUF
    lax.fori_loop(0, nmid, body, 0)
    for j in range(SDEPTH + nmid * NBUF, NBATCH):
        jd = j - SDEPTH
        wait_s(jd, jd % NBUF)
        if jd + NBUF < NBATCH:
            fire_g(jd + NBUF, jd % NBUF)
        wait_g(j, j % NBUF)
        fire_s(j, j % NBUF)
    for j in range(NBATCH - SDEPTH, NBATCH):
        wait_s(j, j % NBUF)

    plsc.subcore_barrier()

    pltpu.sync_copy(agg_sh.at[pl.ds(s * ROWS_PER_SUB, ROWS_PER_SUB)],
                    out_hbm.at[c, pl.ds(s * ROWS_PER_SUB, ROWS_PER_SUB)])


@functools.cache
def _make_sc_scatter():
    return pl.kernel(
        _sc_scatter_body,
        mesh=plsc.VectorSubcoreMesh(core_axis_name="c", subcore_axis_name="s",
                                    num_cores=NCORES, num_subcores=NSUB),
        out_type=jax.ShapeDtypeStruct((NCORES, NPAD, HDIM), jnp.float32),
        scratch_types=(
            [pltpu.VMEM((NBATCH, BATCH), jnp.int32),
             pltpu.VMEM((NBATCH, BATCH), jnp.int32)]
            + [pltpu.VMEM((BATCH, HDIM), jnp.float32)] * NBUF
            + [pltpu.VMEM((ZROWS, HDIM), jnp.float32),
               pltpu.VMEM_SHARED((NPAD, HDIM), jnp.float32),
               pltpu.SemaphoreType.DMA,
               pltpu.SemaphoreType.DMA]
        ),
        compiler_params=pltpu.CompilerParams(use_tc_tiling_on_sc=False),
    )


def _tc_pre_body(h_ref, wi_ref, whh_t_ref, bhh_ref, m_ref, gh_ref):
    h = h_ref[...]
    m = jnp.dot(h, wi_ref[...], preferred_element_type=jnp.float32)
    m_ref[0] = m[:, :HDIM]
    m_ref[1] = m[:, HDIM:]
    gh_ref[...] = (jnp.dot(h, whh_t_ref[...], preferred_element_type=jnp.float32)
                   + bhh_ref[...])


def _tc_pre(h, wi, whh_t, bhh):
    nblk = NNODES // ROW_BLK
    return pl.pallas_call(
        _tc_pre_body,
        grid=(nblk,),
        in_specs=[
            pl.BlockSpec((ROW_BLK, DIM), lambda i: (i, 0)),
            pl.BlockSpec((DIM, DIM), lambda i: (0, 0)),
            pl.BlockSpec((DIM, 3 * DIM), lambda i: (0, 0)),
            pl.BlockSpec((1, 3 * DIM), lambda i: (0, 0)),
        ],
        out_specs=[
            pl.BlockSpec((NCORES, ROW_BLK, HDIM), lambda i: (0, i, 0)),
            pl.BlockSpec((ROW_BLK, 3 * DIM), lambda i: (i, 0)),
        ],
        out_shape=[
            jax.ShapeDtypeStruct((NCORES, NNODES, HDIM), jnp.float32),
            jax.ShapeDtypeStruct((NNODES, 3 * DIM), jnp.float32),
        ],
    )(h, wi, whh_t, bhh)


def _tc_gru_body(agg2_ref, gh_ref, h_ref, wih_t_ref, bih_ref, hout_ref):
    agg = jnp.concatenate([agg2_ref[0], agg2_ref[1]], axis=1)
    gi = (jnp.dot(agg, wih_t_ref[...], preferred_element_type=jnp.float32)
          + bih_ref[...])
    gh = gh_ref[...]
    h = h_ref[...]
    r = jax.nn.sigmoid(gi[:, :DIM] + gh[:, :DIM])
    z = jax.nn.sigmoid(gi[:, DIM:2 * DIM] + gh[:, DIM:2 * DIM])
    n = jnp.tanh(gi[:, 2 * DIM:] + r * gh[:, 2 * DIM:])
    hout_ref[...] = (1.0 - z) * n + z * h


def _tc_gru(agg2, gh, h, wih_t, bih):
    nblk = NNODES // ROW_BLK
    return pl.pallas_call(
        _tc_gru_body,
        grid=(nblk,),
        in_specs=[
            pl.BlockSpec((NCORES, ROW_BLK, HDIM), lambda i: (0, i, 0)),
            pl.BlockSpec((ROW_BLK, 3 * DIM), lambda i: (i, 0)),
            pl.BlockSpec((ROW_BLK, DIM), lambda i: (i, 0)),
            pl.BlockSpec((DIM, 3 * DIM), lambda i: (0, 0)),
            pl.BlockSpec((1, 3 * DIM), lambda i: (0, 0)),
        ],
        out_specs=pl.BlockSpec((ROW_BLK, DIM), lambda i: (i, 0)),
        out_shape=jax.ShapeDtypeStruct((NNODES, DIM), jnp.float32),
    )(agg2, gh, h, wih_t, bih)


def _tc_fin_body(h_ref, fcw_t_ref, fcb_ref, o_ref):
    o_ref[...] = (jnp.dot(jnp.maximum(h_ref[...], 0.0), fcw_t_ref[...],
                          preferred_element_type=jnp.float32)
                  + fcb_ref[...])


def _tc_fin(h, fcw_t, fcb):
    nblk = NNODES // ROW_BLK
    return pl.pallas_call(
        _tc_fin_body,
        grid=(nblk,),
        in_specs=[
            pl.BlockSpec((ROW_BLK, DIM), lambda i: (i, 0)),
            pl.BlockSpec((DIM, DIM), lambda i: (0, 0)),
            pl.BlockSpec((1, DIM), lambda i: (0, 0)),
        ],
        out_specs=pl.BlockSpec((ROW_BLK, DIM), lambda i: (i, 0)),
        out_shape=jax.ShapeDtypeStruct((NNODES, DIM), jnp.float32),
    )(h, fcw_t, fcb)


def kernel(x, edge_index, W, w_ih, w_hh, b_ih, b_hh, fc_w, fc_b):
    src = edge_index[0]
    dst = edge_index[1]
    pad = EPAD - NEDGES
    src_p = jnp.concatenate([src, jnp.zeros((pad,), jnp.int32)])
    dst_p = jnp.concatenate([dst, jnp.full((pad,), NNODES, jnp.int32)])
    edges_p = jnp.stack([src_p, dst_p]).reshape(2, NSUB, NBATCH, BATCH)

    whh_t = w_hh.T
    wih_t = w_ih.T
    fcw_t = fc_w.T
    bhh = b_hh.reshape(1, -1)
    bih = b_ih.reshape(1, -1)
    fcb = fc_b.reshape(1, -1)

    h = x
    for i in range(W.shape[0]):
        m, gh = _tc_pre(h, W[i], whh_t, bhh)
        agg2 = _make_sc_scatter()(m, edges_p)
        h = _tc_gru(agg2, gh, h, wih_t, bih)
    return _tc_fin(h, fcw_t, fcb)


# speedup vs baseline: 4.7302x; 1.0049x over previous
import functools

import jax
import jax.numpy as jnp
from jax import lax
from jax.experimental import pallas as pl
from jax.experimental.pallas import tpu as pltpu
from jax.experimental.pallas import tpu_sc as plsc

NNODES = 10000
DIM = 128
NEDGES = 320000

NCORES = 2
NSUB = 16
HDIM = DIM // NCORES
BATCH = 128
NBATCH = 160
EPAD = NSUB * NBATCH * BATCH
NPAD = 10240
ROWS_PER_SUB = NPAD // NSUB
ZROWS = 64

ROW_BLK = 1000


NBUF = 5
SDEPTH = 3


def _sc_scatter_body(m_hbm, edges_hbm, out_hbm,
                     src_v, dst_v, *rest):
    rows = rest[:NBUF]
    zbuf, agg_sh, gsem, ssem = rest[NBUF:]
    c = lax.axis_index("c")
    s = lax.axis_index("s")

    zero16 = jnp.zeros((16,), jnp.float32)

    def zrow(i, _):
        def zcol(j, _):
            zbuf[i, pl.ds(j * 16, 16)] = zero16
            return 0
        return lax.fori_loop(0, DIM // 16, zcol, 0)

    lax.fori_loop(0, ZROWS, zrow, 0)
    for k in range(ROWS_PER_SUB // ZROWS):
        pltpu.sync_copy(zbuf, agg_sh.at[pl.ds(s * ROWS_PER_SUB + k * ZROWS, ZROWS)])

    pltpu.sync_copy(edges_hbm.at[0, s], src_v)
    pltpu.sync_copy(edges_hbm.at[1, s], dst_v)

    plsc.subcore_barrier()

    mc = m_hbm.at[c]

    def fire_g(j, b):
        pltpu.async_copy(mc.at[src_v.at[j]], rows[b], gsem)

    def wait_g(j, b):
        pltpu.make_async_copy(mc.at[src_v.at[j]], rows[b], gsem).wait()

    def fire_s(j, b):
        pltpu.async_copy(rows[b], agg_sh.at[dst_v.at[j]], ssem, add=True)

    def wait_s(j, b):
        pltpu.make_async_copy(rows[b], agg_sh.at[dst_v.at[j]], ssem).wait()

    def slot(j, full):
        b = j % NBUF
        if full:
            jd = j - SDEPTH
            wait_s(jd, jd % NBUF)
            fire_g(jd + NBUF, jd % NBUF)
        wait_g(j, b)
        fire_s(j, b)

    for b in range(NBUF):
        fire_g(b, b)
    for j in range(SDEPTH):
        slot(j, full=False)

    def body(i, _):
        jbase = SDEPTH + i * NBUF
        for u in range(NBUF):
            j = jbase + u
            b = (SDEPTH + u) % NBUF
            bd = u
            wait_s(j - SDEPTH, bd)
            fire_g(j - SDEPTH + NBUF, bd)
            wait_g(j, b)
            fire_s(j, b)
        return 0

    nmid = (NBATCH - NBUF - SDEPTH) // NBUF
    lax.fori_loop(0, nmid, body, 0)
    for j in range(SDEPTH + nmid * NBUF, NBATCH):
        jd = j - SDEPTH
        wait_s(jd, jd % NBUF)
        if jd + NBUF < NBATCH:
            fire_g(jd + NBUF, jd % NBUF)
        wait_g(j, j % NBUF)
        fire_s(j, j % NBUF)
    for j in range(NBATCH - SDEPTH, NBATCH):
        wait_s(j, j % NBUF)

    plsc.subcore_barrier()

    pltpu.sync_copy(agg_sh.at[pl.ds(s * ROWS_PER_SUB, ROWS_PER_SUB)],
                    out_hbm.at[c, pl.ds(s * ROWS_PER_SUB, ROWS_PER_SUB)])


@functools.cache
def _make_sc_scatter():
    return pl.kernel(
        _sc_scatter_body,
        mesh=plsc.VectorSubcoreMesh(core_axis_name="c", subcore_axis_name="s",
                                    num_cores=NCORES, num_subcores=NSUB),
        out_type=jax.ShapeDtypeStruct((NCORES, NPAD, HDIM), jnp.float32),
        scratch_types=(
            [pltpu.VMEM((NBATCH, BATCH), jnp.int32),
             pltpu.VMEM((NBATCH, BATCH), jnp.int32)]
            + [pltpu.VMEM((BATCH, HDIM), jnp.float32)] * NBUF
            + [pltpu.VMEM((ZROWS, HDIM), jnp.float32),
               pltpu.VMEM_SHARED((NPAD, HDIM), jnp.float32),
               pltpu.SemaphoreType.DMA,
               pltpu.SemaphoreType.DMA]
        ),
        compiler_params=pltpu.CompilerParams(use_tc_tiling_on_sc=False),
    )


def _tc_pre_body(h_ref, wi_ref, whh_t_ref, bhh_ref, m_ref, gh_ref):
    h = h_ref[...]
    m = jnp.dot(h, wi_ref[...], preferred_element_type=jnp.float32)
    m_ref[0] = m[:, :HDIM]
    m_ref[1] = m[:, HDIM:]
    gh_ref[...] = (jnp.dot(h, whh_t_ref[...], preferred_element_type=jnp.float32)
                   + bhh_ref[...])


def _tc_pre(h, wi, whh_t, bhh):
    nblk = NNODES // ROW_BLK
    return pl.pallas_call(
        _tc_pre_body,
        grid=(nblk,),
        in_specs=[
            pl.BlockSpec((ROW_BLK, DIM), lambda i: (i, 0)),
            pl.BlockSpec((DIM, DIM), lambda i: (0, 0)),
            pl.BlockSpec((DIM, 3 * DIM), lambda i: (0, 0)),
            pl.BlockSpec((1, 3 * DIM), lambda i: (0, 0)),
        ],
        out_specs=[
            pl.BlockSpec((NCORES, ROW_BLK, HDIM), lambda i: (0, i, 0)),
            pl.BlockSpec((ROW_BLK, 3 * DIM), lambda i: (i, 0)),
        ],
        out_shape=[
            jax.ShapeDtypeStruct((NCORES, NNODES, HDIM), jnp.float32),
            jax.ShapeDtypeStruct((NNODES, 3 * DIM), jnp.float32),
        ],
    )(h, wi, whh_t, bhh)


def _tc_gru_body(agg2_ref, gh_ref, h_ref, wih_t_ref, bih_ref, hout_ref):
    agg = jnp.concatenate([agg2_ref[0], agg2_ref[1]], axis=1)
    gi = (jnp.dot(agg, wih_t_ref[...], preferred_element_type=jnp.float32)
          + bih_ref[...])
    gh = gh_ref[...]
    h = h_ref[...]
    r = jax.nn.sigmoid(gi[:, :DIM] + gh[:, :DIM])
    z = jax.nn.sigmoid(gi[:, DIM:2 * DIM] + gh[:, DIM:2 * DIM])
    n = jnp.tanh(gi[:, 2 * DIM:] + r * gh[:, 2 * DIM:])
    hout_ref[...] = (1.0 - z) * n + z * h


def _tc_gru(agg2, gh, h, wih_t, bih):
    nblk = NNODES // ROW_BLK
    return pl.pallas_call(
        _tc_gru_body,
        grid=(nblk,),
        in_specs=[
            pl.BlockSpec((NCORES, ROW_BLK, HDIM), lambda i: (0, i, 0)),
            pl.BlockSpec((ROW_BLK, 3 * DIM), lambda i: (i, 0)),
            pl.BlockSpec((ROW_BLK, DIM), lambda i: (i, 0)),
            pl.BlockSpec((DIM, 3 * DIM), lambda i: (0, 0)),
            pl.BlockSpec((1, 3 * DIM), lambda i: (0, 0)),
        ],
        out_specs=pl.BlockSpec((ROW_BLK, DIM), lambda i: (i, 0)),
        out_shape=jax.ShapeDtypeStruct((NNODES, DIM), jnp.float32),
    )(agg2, gh, h, wih_t, bih)


def _tc_fin_body(h_ref, fcw_t_ref, fcb_ref, o_ref):
    o_ref[...] = (jnp.dot(jnp.maximum(h_ref[...], 0.0), fcw_t_ref[...],
                          preferred_element_type=jnp.float32)
                  + fcb_ref[...])


def _tc_fin(h, fcw_t, fcb):
    nblk = NNODES // ROW_BLK
    return pl.pallas_call(
        _tc_fin_body,
        grid=(nblk,),
        in_specs=[
            pl.BlockSpec((ROW_BLK, DIM), lambda i: (i, 0)),
            pl.BlockSpec((DIM, DIM), lambda i: (0, 0)),
            pl.BlockSpec((1, DIM), lambda i: (0, 0)),
        ],
        out_specs=pl.BlockSpec((ROW_BLK, DIM), lambda i: (i, 0)),
        out_shape=jax.ShapeDtypeStruct((NNODES, DIM), jnp.float32),
    )(h, fcw_t, fcb)


def kernel(x, edge_index, W, w_ih, w_hh, b_ih, b_hh, fc_w, fc_b):
    src = edge_index[0]
    dst = edge_index[1]
    pad = EPAD - NEDGES
    src_p = jnp.concatenate([src, jnp.zeros((pad,), jnp.int32)])
    dst_p = jnp.concatenate([dst, jnp.full((pad,), NNODES, jnp.int32)])
    edges_p = jnp.stack([src_p, dst_p]).reshape(2, NSUB, NBATCH, BATCH)

    whh_t = w_hh.T
    wih_t = w_ih.T
    fcw_t = fc_w.T
    bhh = b_hh.reshape(1, -1)
    bih = b_ih.reshape(1, -1)
    fcb = fc_b.reshape(1, -1)

    h = x
    for i in range(W.shape[0]):
        m, gh = _tc_pre(h, W[i], whh_t, bhh)
        agg2 = _make_sc_scatter()(m, edges_p)
        h = _tc_gru(agg2, gh, h, wih_t, bih)
    return _tc_fin(h, fcw_t, fcb)
